# baseline (speedup 1.0000x reference)
import functools

import jax
import jax.numpy as jnp
from jax import lax
from jax.experimental import pallas as pl
from jax.experimental.pallas import tpu as pltpu
from jax.experimental.pallas import tpu_sc as plsc

D = 1024
B = 4 * 4096
NW = 32
B_PER_W = B // NW
CHUNK = 16
N_CHUNKS = B_PER_W // CHUNK


NBUF = 6


def _emb_kernel(table_hbm, idx_hbm, out_hbm, idx_v, buf0, buf1, buf2, buf3, buf4, buf5,
                gsem0, gsem1, gsem2, gsem3, gsem4, gsem5,
                ssem0, ssem1, ssem2, ssem3, ssem4, ssem5):
    wid = lax.axis_index("s") * 2 + lax.axis_index("c")
    pltpu.sync_copy(idx_hbm.at[wid, pl.ds(0, 8)], idx_v.at[pl.ds(0, 8)])

    bufs = (buf0, buf1, buf2, buf3, buf4, buf5)
    gsems = (gsem0, gsem1, gsem2, gsem3, gsem4, gsem5)
    ssems = (ssem0, ssem1, ssem2, ssem3, ssem4, ssem5)

    for j in range(NBUF - 1):
        pltpu.async_copy(table_hbm.at[idx_v.at[j]], bufs[j], gsems[j])
    pltpu.sync_copy(idx_hbm.at[wid, pl.ds(8, N_CHUNKS - 8)],
                    idx_v.at[pl.ds(8, N_CHUNKS - 8)])
    A = NBUF - 1
    for i in range(N_CHUNKS):
        if i + A < N_CHUNKS:
            b = (i + A) % NBUF
            if i >= 1:
                pltpu.make_async_copy(bufs[b], out_hbm.at[pl.ds(0, CHUNK)], ssems[b]).wait()
            pltpu.async_copy(table_hbm.at[idx_v.at[i + A]], bufs[b], gsems[b])
        cur = i % NBUF
        pltpu.make_async_copy(table_hbm.at[idx_v.at[i]], bufs[cur], gsems[cur]).wait()
        pltpu.async_copy(bufs[cur], out_hbm.at[pl.ds((i * NW + wid) * CHUNK, CHUNK)], ssems[cur])
    for i in range(N_CHUNKS - NBUF, N_CHUNKS):
        b = i % NBUF
        pltpu.make_async_copy(bufs[b], out_hbm.at[pl.ds(0, CHUNK)], ssems[b]).wait()


@jax.jit
def _run(ids_grp, wte):
    mesh = plsc.VectorSubcoreMesh(core_axis_name="c", subcore_axis_name="s")
    k = functools.partial(
        pl.kernel,
        mesh=mesh,
        out_type=jax.ShapeDtypeStruct((B, D), jnp.float32),
        scratch_types=[
            pltpu.VMEM((N_CHUNKS, CHUNK), jnp.int32),
        ] + [pltpu.VMEM((CHUNK, D), jnp.float32)] * 6 + [pltpu.SemaphoreType.DMA] * 12,
    )(_emb_kernel)
    return k(wte, ids_grp)


def kernel(input_ids, wte):
    ids_grp = input_ids.reshape(N_CHUNKS, NW, CHUNK).transpose(1, 0, 2).astype(jnp.int32)
    out = _run(ids_grp, wte)
    return out.reshape(input_ids.shape + (D,))
